# baseline (speedup 1.0000x reference)
import functools

import jax
import jax.numpy as jnp
from jax import lax
from jax.experimental import pallas as pl
from jax.experimental.pallas import tpu as pltpu
from jax.experimental.pallas import tpu_sc as plsc

_B, _H, _W, _CIN = 4, 512, 512, 3
_GH, _GW, _GD, _GC = 16, 16, 8, 12
_COUT = _GC // (_CIN + 1)

_ROWS_PER_TEC = 64
_CHUNK = 8
_CPIX = _CHUNK * _W


def _conv_body(img_ref, w_ref, b_ref, out_ref):
    f32 = jnp.float32
    padded = []
    for ci in range(_CIN):
        ich = img_ref[0, ci]
        hp = jnp.concatenate(
            [jnp.zeros((_H, 128), f32), ich, jnp.zeros((_H, 128), f32)], axis=1)
        vp = jnp.concatenate(
            [jnp.zeros((8, _W + 256), f32), hp, jnp.zeros((8, _W + 256), f32)],
            axis=0)
        padded.append(vp)
    for co in range(_CIN):
        acc = jnp.full((_H, _W), b_ref[co], f32)
        for dy in range(3):
            for dx in range(3):
                for ci in range(_CIN):
                    w = w_ref[dy, dx, ci, co]
                    acc = acc + w * jax.lax.slice(
                        padded[ci], (8 + dy - 1, 128 + dx - 1),
                        (8 + dy - 1 + _H, 128 + dx - 1 + _W))
        out_ref[0, co] = acc


def _run_conv(image_t, W_conv, b_conv):
    return pl.pallas_call(
        _conv_body,
        grid=(_B,),
        in_specs=[
            pl.BlockSpec((1, _CIN, _H, _W), lambda b: (b, 0, 0, 0)),
            pl.BlockSpec(memory_space=pltpu.SMEM),
            pl.BlockSpec(memory_space=pltpu.SMEM),
        ],
        out_specs=pl.BlockSpec((1, _CIN, _H, _W), lambda b: (b, 0, 0, 0)),
        out_shape=jax.ShapeDtypeStruct((_B, _CIN, _H, _W), jnp.float32),
    )(image_t, W_conv, b_conv)


def _sc_body(grid_hbm, guide_hbm, img_hbm, out_hbm,
             grid_v, guide_v, img_v, out_v):
    f32 = jnp.float32
    i32 = jnp.int32
    cid = lax.axis_index("c")
    sid = lax.axis_index("s")
    wid = sid * 2 + cid
    batch = wid // 8
    rowbase = (wid % 8) * _ROWS_PER_TEC
    _NPIX = _H * _W

    pltpu.sync_copy(grid_hbm.at[pl.ds(batch * (_GH * _GW * _GD * _GC),
                                      _GH * _GW * _GD * _GC)], grid_v)

    def chunk_body(rc, carry):
        row0 = rowbase + rc * _CHUNK
        off = batch * _NPIX + row0 * _W
        pltpu.sync_copy(guide_hbm.at[pl.ds(off, _CPIX)], guide_v)
        for ci in range(_CIN):
            pltpu.sync_copy(
                img_hbm.at[pl.ds((batch * _CIN + ci - batch) * _NPIX + off,
                                 _CPIX)],
                img_v.at[pl.ds(ci * _CPIX, _CPIX)])

        def p_body(p, c2):
            i = lax.shift_right_logical(p, 5)
            j = lax.bitwise_and(p, 31)
            row = row0 + i
            fyi = lax.shift_right_arithmetic(row - 16, 5)
            wy1 = (row.astype(f32) + 0.5) * (1.0 / 32.0) - 0.5 - fyi.astype(f32)
            y0 = jnp.clip(fyi, 0, _GH - 1)
            y1 = jnp.clip(fyi + 1, 0, _GH - 1)
            colv = j * 16 + lax.iota(i32, 16)
            fxv = lax.shift_right_arithmetic(colv - 16, 5)
            wx1 = (colv.astype(f32) + 0.5) * (1.0 / 32.0) - 0.5 - fxv.astype(f32)
            x0 = jnp.clip(fxv, 0, _GW - 1)
            x1 = jnp.clip(fxv + 1, 0, _GW - 1)
            g = guide_v[pl.ds(p * 16, 16)]
            t = jnp.clip(g, 0.0, 1.0) * float(_GD) - 0.5
            ti = t.astype(i32)
            fzv = jnp.where(t < ti.astype(f32), ti - 1, ti)
            wz1 = t - fzv.astype(f32)
            z0 = jnp.clip(fzv, 0, _GD - 1)
            z1 = jnp.clip(fzv + 1, 0, _GD - 1)

            wys = ((1.0 - wy1), wy1)
            ybs = (y0 * (_GW * _GD * _GC), y1 * (_GW * _GD * _GC))
            wxs = ((1.0 - wx1), wx1)
            xbs = (x0 * (_GD * _GC), x1 * (_GD * _GC))
            wzs = ((1.0 - wz1), wz1)
            zbs = (z0 * _GC, z1 * _GC)

            coeff = [None] * _GC
            for a in range(2):
                for b in range(2):
                    wxy = wys[a] * wxs[b]
                    bxy = ybs[a] + xbs[b]
                    for d in range(2):
                        w = wxy * wzs[d]
                        base = bxy + zbs[d]
                        for c in range(_GC):
                            gv = plsc.load_gather(grid_v, [base + c])
                            if coeff[c] is None:
                                coeff[c] = w * gv
                            else:
                                coeff[c] = coeff[c] + w * gv

            for co in range(_COUT):
                res = coeff[(_CIN + 1) * co + _CIN]
                for ci in range(_CIN):
                    imgv = img_v[pl.ds(ci * _CPIX + p * 16, 16)]
                    res = res + coeff[(_CIN + 1) * co + ci] * imgv
                out_v[pl.ds(co * _CPIX + p * 16, 16)] = res
            return c2

        lax.fori_loop(0, _CPIX // 16, p_body, 0)
        for co in range(_COUT):
            pltpu.sync_copy(
                out_v.at[pl.ds(co * _CPIX, _CPIX)],
                out_hbm.at[pl.ds((batch * _COUT + co - batch) * _NPIX + off,
                                 _CPIX)])
        return carry

    lax.fori_loop(0, _ROWS_PER_TEC // _CHUNK, chunk_body, 0)


def _run_sc(grid_flat, guide_flat, img_flat):
    mesh = plsc.VectorSubcoreMesh(core_axis_name="c", subcore_axis_name="s")
    f = functools.partial(
        pl.kernel,
        mesh=mesh,
        compiler_params=pltpu.CompilerParams(needs_layout_passes=False),
        out_type=jax.ShapeDtypeStruct((_B * _COUT * _H * _W,), jnp.float32),
        scratch_types=[
            pltpu.VMEM((_GH * _GW * _GD * _GC,), jnp.float32),
            pltpu.VMEM((_CPIX,), jnp.float32),
            pltpu.VMEM((_CIN * _CPIX,), jnp.float32),
            pltpu.VMEM((_COUT * _CPIX,), jnp.float32),
        ],
    )(_sc_body)
    return f(grid_flat, guide_flat, img_flat)


def kernel(grid_th, guide_th, image_th, W_conv, b_conv):
    image_t = jnp.transpose(image_th, (0, 3, 1, 2))
    conv_t = _run_conv(image_t, W_conv, b_conv)
    grid_flat = grid_th.reshape(-1)
    guide_flat = guide_th.reshape(-1)
    img_flat = conv_t.reshape(-1)
    out_flat = _run_sc(grid_flat, guide_flat, img_flat)
    out_t = out_flat.reshape(_B, _COUT, _H, _W)
    return jnp.transpose(out_t, (0, 2, 3, 1))
